# baseline (speedup 1.0000x reference)
import functools
import jax
import jax.numpy as jnp
from jax import lax
from jax.experimental import pallas as pl
from jax.experimental.pallas import tpu as pltpu
from jax.experimental.pallas import tpu_sc as plsc

NN = 10000
NE = 320000
NH = 4
SC_CORES = 2
SC_SUBCORES = 16
NSUB = SC_CORES * SC_SUBCORES

CE = 2000

_mesh = plsc.VectorSubcoreMesh(
    core_axis_name="c", subcore_axis_name="s",
    num_cores=SC_CORES, num_subcores=SC_SUBCORES)


def _lrelu(x):
    return jnp.where(x > 0, x, 0.2 * x)


def _prep1_body(h_ref, w_ref, b_ref, ad_ref, ab_ref, as_ref,
                wh_ref, td_ref, ts_ref, msp_ref):
    wh = jnp.dot(h_ref[...], w_ref[...], preferred_element_type=jnp.float32)
    wh = wh + b_ref[...]
    wh_ref[...] = wh
    td = jnp.dot(wh, ad_ref[...], preferred_element_type=jnp.float32) + ab_ref[...]
    ts = jnp.dot(wh, as_ref[...], preferred_element_type=jnp.float32)
    td_ref[...] = td
    ts_ref[...] = ts
    m = jnp.max(ts, axis=0)
    msp_ref[...] = jnp.broadcast_to(m[:, None], (NH, 16))


def _prep1(h, wcatT, bcat, Ad, Ab, As, dout):
    return pl.pallas_call(
        _prep1_body,
        out_shape=(
            jax.ShapeDtypeStruct((NN, dout), jnp.float32),
            jax.ShapeDtypeStruct((NN, NH), jnp.float32),
            jax.ShapeDtypeStruct((NN, NH), jnp.float32),
            jax.ShapeDtypeStruct((NH, 16), jnp.float32),
        ),
    )(h, wcatT, bcat.reshape(1, dout), Ad, Ab.reshape(1, NH), As)


def _prep2_body(o_ref, w_ref, b_ref, ad_ref, ab_ref, as_ref,
                wh_ref, td_ref, ts_ref, msp_ref):
    x = jnp.concatenate([o_ref[:NN, :], o_ref[NN:, :]], axis=1)
    x = jnp.where(x > 0, x, jnp.exp(x) - 1.0)
    wh = jnp.dot(x, w_ref[...], preferred_element_type=jnp.float32) + b_ref[...]
    wh_ref[...] = wh
    td = jnp.dot(wh, ad_ref[...], preferred_element_type=jnp.float32) + ab_ref[...]
    ts = jnp.dot(wh, as_ref[...], preferred_element_type=jnp.float32)
    td_ref[...] = td
    ts_ref[...] = ts
    m = jnp.max(ts, axis=0)
    msp_ref[...] = jnp.broadcast_to(m[:, None], (NH, 16))


def _prep2(out1, wcatT, bcat, Ad, Ab, As, dout):
    return pl.pallas_call(
        _prep2_body,
        out_shape=(
            jax.ShapeDtypeStruct((NN, dout), jnp.float32),
            jax.ShapeDtypeStruct((NN, NH), jnp.float32),
            jax.ShapeDtypeStruct((NN, NH), jnp.float32),
            jax.ShapeDtypeStruct((NH, 16), jnp.float32),
        ),
    )(out1, wcatT, bcat.reshape(1, dout), Ad, Ab.reshape(1, NH), As)


def _final_body(o_ref, fcw_ref, fcb_ref, out_ref):
    x = (o_ref[:NN, 0:32] + o_ref[:NN, 32:64]
         + o_ref[NN:, 0:32] + o_ref[NN:, 32:64]) * 0.25
    x = x - jnp.max(x, axis=1, keepdims=True)
    ex = jnp.exp(x)
    x = ex / jnp.sum(ex, axis=1, keepdims=True)
    hg = jnp.mean(x, axis=0, keepdims=True)
    out_ref[...] = jnp.dot(hg, fcw_ref[...], preferred_element_type=jnp.float32) + fcb_ref[...]


def _final(out2, fcW, fcb):
    return pl.pallas_call(
        _final_body,
        out_shape=jax.ShapeDtypeStruct((1, 32), jnp.float32),
    )(out2, fcW.T, fcb.reshape(1, 32))


def _scA_body(src_hbm, dst_hbm, td_hbm, ts_hbm, msp_hbm,
              p_hbm, sp_hbm,
              td_v, ts_v, msp_v, src_v, dst_v, pall_v, sidx_v, s_sh):
    c = lax.axis_index("c")
    s = lax.axis_index("s")

    def _zb(i, _):
        pall_v[pl.ds(pl.multiple_of(i * 16, 16), 16)] = jnp.zeros((16,), jnp.float32)
        return 0
    lax.fori_loop(0, 4000 // 16, _zb, 0)

    @pl.when(s < 10)
    def _zero():
        pltpu.sync_copy(pall_v.at[pl.ds(0, 4000)],
                        s_sh.at[pl.ds(s * 4000, 4000)])

    pltpu.sync_copy(td_hbm, td_v)
    pltpu.sync_copy(ts_hbm, ts_v)
    pltpu.sync_copy(msp_hbm, msp_v)
    plsc.subcore_barrier()

    wid = c * SC_SUBCORES + s
    lane = lax.iota(jnp.int32, 16)

    def _chunk(k, _):
        base = wid * (NE // NSUB) + k * CE
        pltpu.sync_copy(src_hbm.at[pl.ds(base, CE)], src_v)
        pltpu.sync_copy(dst_hbm.at[pl.ds(base, CE)], dst_v)

        @plsc.parallel_loop(0, CE // 16, 1, unroll=2)
        def _vec(i):
            off = pl.multiple_of(i * 16, 16)
            d16 = dst_v[pl.ds(off, 16)]
            s16 = src_v[pl.ds(off, 16)]
            rows = off + lane
            for hh in range(NH):
                td16 = plsc.load_gather(td_v, [d16 * NH + hh])
                ts16 = plsc.load_gather(ts_v, [s16 * NH + hh])
                mh = msp_v[pl.ds(hh * 16, 16)]
                e = _lrelu(td16 + ts16)
                mp = _lrelu(td16 + mh)
                p = jnp.exp(e - mp)
                plsc.store_scatter(pall_v, [rows * NH + hh], p)
                plsc.store_scatter(sidx_v, [rows * NH + hh], d16 * NH + hh)

        pltpu.sync_copy(pall_v, p_hbm.at[pl.ds(base * NH, CE * NH)])
        pltpu.sync_copy(pall_v, s_sh.at[sidx_v], add=True)
        return 0
    lax.fori_loop(0, NE // NSUB // CE, _chunk, 0)

    plsc.subcore_barrier()

    @pl.when(s < 10)
    def _writeout():
        pltpu.sync_copy(s_sh.at[pl.ds(s * 4000, 4000)],
                        sp_hbm.at[pl.ds(c * (NN * NH) + s * 4000, 4000)])


_scA = functools.partial(
    pl.kernel,
    out_type=(
        jax.ShapeDtypeStruct((NE * NH,), jnp.float32),
        jax.ShapeDtypeStruct((SC_CORES * NN * NH,), jnp.float32),
    ),
    mesh=_mesh,
    compiler_params=pltpu.CompilerParams(
        needs_layout_passes=False, use_tc_tiling_on_sc=False),
    scratch_types=[
        pltpu.VMEM((NN * NH,), jnp.float32),
        pltpu.VMEM((NN * NH,), jnp.float32),
        pltpu.VMEM((NH * 16,), jnp.float32),
        pltpu.VMEM((CE,), jnp.int32),
        pltpu.VMEM((CE,), jnp.int32),
        pltpu.VMEM((CE * NH,), jnp.float32),
        pltpu.VMEM((CE * NH,), jnp.int32),
        pltpu.VMEM_SHARED((NN * NH,), jnp.float32),
    ],
)(_scA_body)


def _scN_body(dst_hbm, p_hbm, sp_hbm,
              w_hbm,
              r_v, sA_v, sB_v, dst_v, p_v):
    c = lax.axis_index("c")
    s = lax.axis_index("s")
    lane = lax.iota(jnp.int32, 16)

    for k in range(10):
        pltpu.sync_copy(sp_hbm.at[pl.ds(k * 4000, 4000)], sA_v)
        pltpu.sync_copy(sp_hbm.at[pl.ds(NN * NH + k * 4000, 4000)], sB_v)

        def _rv(i, _, k=k):
            off = pl.multiple_of(i * 16, 16)
            a = sA_v[pl.ds(off, 16)]
            b = sB_v[pl.ds(off, 16)]
            r_v[pl.ds(k * 4000 + off, 16)] = 1.0 / (a + b + 1e-16)
            return 0
        lax.fori_loop(0, 250, _rv, 0)

    wid = c * SC_SUBCORES + s

    def _chunk(k, _):
        base = wid * (NE // NSUB) + k * CE
        pltpu.sync_copy(dst_hbm.at[pl.ds(base, CE)], dst_v)
        pltpu.sync_copy(p_hbm.at[pl.ds(base * NH, CE * NH)], p_v)

        @plsc.parallel_loop(0, CE // 16, 1, unroll=2)
        def _vec(i):
            off = pl.multiple_of(i * 16, 16)
            d16 = dst_v[pl.ds(off, 16)]
            e16 = off + lane
            for hh in range(NH):
                p16 = plsc.load_gather(p_v, [e16 * NH + hh])
                r16 = plsc.load_gather(r_v, [d16 * NH + hh])
                plsc.store_scatter(p_v, [e16 * NH + hh], p16 * r16)

        pltpu.sync_copy(p_v, w_hbm.at[pl.ds(base * NH, CE * NH)])
        return 0
    lax.fori_loop(0, NE // NSUB // CE, _chunk, 0)


_scN = functools.partial(
    pl.kernel,
    out_type=jax.ShapeDtypeStruct((NE * NH,), jnp.float32),
    mesh=_mesh,
    compiler_params=pltpu.CompilerParams(
        needs_layout_passes=False, use_tc_tiling_on_sc=False),
    scratch_types=[
        pltpu.VMEM((NN * NH,), jnp.float32),
        pltpu.VMEM((4000,), jnp.float32),
        pltpu.VMEM((4000,), jnp.float32),
        pltpu.VMEM((CE,), jnp.int32),
        pltpu.VMEM((CE * NH,), jnp.float32),
    ],
)(_scN_body)


CB = 400
SB = 80


def _scB_body(layer, src_hbm, dst_hbm, w_hbm, wh_hbm,
              out_hbm,
              src_v, dst_v, w_v, w2_v, gA_v, gB_v, sA_v, sB_v, out_sh,
              gsemA, gsemB, ssemA, ssemB):
    c = lax.axis_index("c")
    s = lax.axis_index("s")
    lane = lax.iota(jnp.int32, 16)
    two_c = 2 * c
    fo = 128 if layer == 1 else 64
    gbuf = (gA_v, gB_v)
    sbuf = (gA_v, gB_v) if layer == 1 else (sA_v, sB_v)
    gsem = (gsemA, gsemB)
    ssem = (ssemA, ssemB)

    @plsc.parallel_loop(0, SB, 1, unroll=4)
    def _zg(i):
        for f in range(fo // 16):
            sbuf[0][i, pl.ds(f * 16, 16)] = jnp.zeros((16,), jnp.float32)
    for t in range(625 // SB):
        pltpu.sync_copy(sbuf[0], out_sh.at[pl.ds(s * 625 + t * SB, SB)])
    pltpu.sync_copy(sbuf[0].at[pl.ds(0, 625 % SB)],
                    out_sh.at[pl.ds(s * 625 + (625 // SB) * SB, 625 % SB)])
    plsc.subcore_barrier()

    nt = CB // SB

    def _chunk(k, _):
        eb = s * (NE // SC_SUBCORES) + k * CB
        pltpu.sync_copy(src_hbm.at[pl.ds(eb, CB)], src_v)
        pltpu.sync_copy(w_hbm.at[pl.ds(eb * NH, CB * NH)], w_v)
        for t in range(nt):
            pltpu.sync_copy(dst_hbm.at[pl.ds(eb + t * SB, SB)], dst_v.at[t])

        if layer == 1:
            @plsc.parallel_loop(0, CB // 16, 1, unroll=2)
            def _adj(i):
                off = pl.multiple_of(i * 16, 16)
                src_v[pl.ds(off, 16)] = src_v[pl.ds(off, 16)] + c * NN

        @plsc.parallel_loop(0, CB // 16, 1, unroll=2)
        def _wv(i):
            off = pl.multiple_of(i * 16, 16)
            e16 = off + lane
            w0 = plsc.load_gather(w_v, [e16 * NH + two_c])
            w1 = plsc.load_gather(w_v, [e16 * NH + two_c + 1])
            plsc.store_scatter(w2_v, [e16 * 2], w0)
            plsc.store_scatter(w2_v, [e16 * 2 + 1], w1)

        def _issue_gather(t):
            p = t % 2
            return pltpu.async_copy(
                wh_hbm.at[src_v.at[pl.ds(t * SB, SB)]], gbuf[p], gsem[p])

        g_desc = _issue_gather(0)
        s_desc = [None, None]
        for t in range(nt):
            p = t % 2
            if t + 1 < nt:
                if layer == 1 and s_desc[1 - p] is not None:
                    s_desc[1 - p].wait()
                    s_desc[1 - p] = None
                g_next = _issue_gather(t + 1)
            else:
                g_next = None
            g_desc.wait()
            if layer == 2 and s_desc[p] is not None:
                s_desc[p].wait()
                s_desc[p] = None

            @plsc.parallel_loop(0, SB, 1, unroll=4)
            def _scale(e, t=t, p=p):
                wv = w2_v[pl.ds(2 * (t * SB + e), 16)]
                w0 = wv[0]
                w1 = wv[1]
                if layer == 1:
                    for f in range(8):
                        mf = w0 if f < 4 else w1
                        gbuf[p][e, pl.ds(f * 16, 16)] = (
                            gbuf[p][e, pl.ds(f * 16, 16)] * mf)
                else:
                    cbase = pl.multiple_of(64 * c, 16)
                    for f in range(4):
                        mf = w0 if f < 2 else w1
                        sbuf[p][e, pl.ds(f * 16, 16)] = (
                            gbuf[p][e, pl.ds(cbase + f * 16, 16)] * mf)

            s_desc[p] = pltpu.async_copy(
                sbuf[p], out_sh.at[dst_v.at[t]], ssem[p], add=True)
            g_desc = g_next
        for p in range(2):
            if s_desc[p] is not None:
                s_desc[p].wait()
        return 0
    lax.fori_loop(0, NE // SC_SUBCORES // CB, _chunk, 0)

    plsc.subcore_barrier()

    @pl.when(s < 10)
    def _writeout():
        pltpu.sync_copy(out_sh.at[pl.ds(s * 1000, 1000)],
                        out_hbm.at[pl.ds(c * NN + s * 1000, 1000)])


def _make_scB(layer):
    fo = 128 if layer == 1 else 64
    so = 16 if layer == 1 else SB
    return functools.partial(
        pl.kernel,
        out_type=jax.ShapeDtypeStruct((2 * NN, fo), jnp.float32),
        mesh=_mesh,
        compiler_params=pltpu.CompilerParams(
            needs_layout_passes=False, use_tc_tiling_on_sc=False),
        scratch_types=[
            pltpu.VMEM((CB,), jnp.int32),
            pltpu.VMEM((CB // SB, SB), jnp.int32),
            pltpu.VMEM((CB * NH,), jnp.float32),
            pltpu.VMEM((CB * 2 + 16,), jnp.float32),
            pltpu.VMEM((SB, 128), jnp.float32),
            pltpu.VMEM((SB, 128), jnp.float32),
            pltpu.VMEM((so, 64), jnp.float32),
            pltpu.VMEM((so, 64), jnp.float32),
            pltpu.VMEM_SHARED((NN, fo), jnp.float32),
            pltpu.SemaphoreType.DMA,
            pltpu.SemaphoreType.DMA,
            pltpu.SemaphoreType.DMA,
            pltpu.SemaphoreType.DMA,
        ],
    )(functools.partial(_scB_body, layer))


_scB1 = _make_scB(1)
_scB2 = _make_scB(2)


def _make_block_A(Aw, hdim):
    z = jnp.zeros((NH, hdim, NH), jnp.float32)
    idx = jnp.arange(NH)
    z = z.at[idx, :, idx].set(Aw[:, :hdim])
    return z.reshape(NH * hdim, NH)


def _make_block_A2(Aw, hdim):
    z = jnp.zeros((NH, hdim, NH), jnp.float32)
    idx = jnp.arange(NH)
    z = z.at[idx, :, idx].set(Aw[:, hdim:])
    return z.reshape(NH * hdim, NH)


@jax.jit
def kernel(h, edge_index, W1, b1, A1w, A1b, W2, b2, A2w, A2b, fcW, fcb):
    src = edge_index[0]
    dst = edge_index[1]

    w1catT = W1.reshape(NH * 64, 128).T
    b1cat = b1.reshape(NH * 64)
    Ad1 = _make_block_A(A1w, 64)
    As1 = _make_block_A2(A1w, 64)
    w2catT = W2.reshape(NH * 32, NH * 64).T
    b2cat = b2.reshape(NH * 32)
    Ad2 = _make_block_A(A2w, 32)
    As2 = _make_block_A2(A2w, 32)

    wh1, td1, ts1, msp1 = _prep1(h, w1catT, b1cat, Ad1, A1b, As1, 256)
    p1, sp1 = _scA(src, dst, td1.reshape(-1), ts1.reshape(-1),
                   msp1.reshape(-1))
    wh1cat = jnp.concatenate([wh1[:, :128], wh1[:, 128:]], axis=0)
    w1 = _scN(dst, p1, sp1)
    out1cat = _scB1(src, dst, w1, wh1cat)

    wh2, td2, ts2, msp2 = _prep2(out1cat, w2catT, b2cat, Ad2, A2b, As2, 128)
    p2, sp2 = _scA(src, dst, td2.reshape(-1), ts2.reshape(-1),
                   msp2.reshape(-1))
    w2 = _scN(dst, p2, sp2)
    out2cat = _scB2(src, dst, w2, wh2)

    return _final(out2cat, fcW, fcb)


# speedup vs baseline: 1.0272x; 1.0272x over previous
import functools
import jax
import jax.numpy as jnp
from jax import lax
from jax.experimental import pallas as pl
from jax.experimental.pallas import tpu as pltpu
from jax.experimental.pallas import tpu_sc as plsc

NN = 10000
NE = 320000
NH = 4
SC_CORES = 2
SC_SUBCORES = 16
NSUB = SC_CORES * SC_SUBCORES

CE = 2000

_mesh = plsc.VectorSubcoreMesh(
    core_axis_name="c", subcore_axis_name="s",
    num_cores=SC_CORES, num_subcores=SC_SUBCORES)


def _lrelu(x):
    return jnp.where(x > 0, x, 0.2 * x)


def _prep1_body(h_ref, w_ref, b_ref, ad_ref, ab_ref, as_ref,
                wh_ref, td_ref, ts_ref, msp_ref):
    wh = jnp.dot(h_ref[...], w_ref[...], preferred_element_type=jnp.float32)
    wh = wh + b_ref[...]
    wh_ref[...] = wh
    td = jnp.dot(wh, ad_ref[...], preferred_element_type=jnp.float32) + ab_ref[...]
    ts = jnp.dot(wh, as_ref[...], preferred_element_type=jnp.float32)
    td_ref[...] = td
    ts_ref[...] = ts
    m = jnp.max(ts, axis=0)
    msp_ref[...] = jnp.broadcast_to(m[:, None], (NH, 16))


def _prep1(h, wcatT, bcat, Ad, Ab, As, dout):
    return pl.pallas_call(
        _prep1_body,
        out_shape=(
            jax.ShapeDtypeStruct((NN, dout), jnp.float32),
            jax.ShapeDtypeStruct((NN, NH), jnp.float32),
            jax.ShapeDtypeStruct((NN, NH), jnp.float32),
            jax.ShapeDtypeStruct((NH, 16), jnp.float32),
        ),
    )(h, wcatT, bcat.reshape(1, dout), Ad, Ab.reshape(1, NH), As)


def _prep2_body(o_ref, w_ref, b_ref, ad_ref, ab_ref, as_ref,
                wh_ref, td_ref, ts_ref, msp_ref):
    x = jnp.concatenate([o_ref[:NN, :], o_ref[NN:, :]], axis=1)
    x = jnp.where(x > 0, x, jnp.exp(x) - 1.0)
    wh = jnp.dot(x, w_ref[...], preferred_element_type=jnp.float32) + b_ref[...]
    wh_ref[...] = wh
    td = jnp.dot(wh, ad_ref[...], preferred_element_type=jnp.float32) + ab_ref[...]
    ts = jnp.dot(wh, as_ref[...], preferred_element_type=jnp.float32)
    td_ref[...] = td
    ts_ref[...] = ts
    m = jnp.max(ts, axis=0)
    msp_ref[...] = jnp.broadcast_to(m[:, None], (NH, 16))


def _prep2(out1, wcatT, bcat, Ad, Ab, As, dout):
    return pl.pallas_call(
        _prep2_body,
        out_shape=(
            jax.ShapeDtypeStruct((NN, dout), jnp.float32),
            jax.ShapeDtypeStruct((NN, NH), jnp.float32),
            jax.ShapeDtypeStruct((NN, NH), jnp.float32),
            jax.ShapeDtypeStruct((NH, 16), jnp.float32),
        ),
    )(out1, wcatT, bcat.reshape(1, dout), Ad, Ab.reshape(1, NH), As)


def _final_body(o_ref, fcw_ref, fcb_ref, out_ref):
    x = (o_ref[:NN, 0:32] + o_ref[:NN, 32:64]
         + o_ref[NN:, 0:32] + o_ref[NN:, 32:64]) * 0.25
    x = x - jnp.max(x, axis=1, keepdims=True)
    ex = jnp.exp(x)
    x = ex / jnp.sum(ex, axis=1, keepdims=True)
    hg = jnp.mean(x, axis=0, keepdims=True)
    out_ref[...] = jnp.dot(hg, fcw_ref[...], preferred_element_type=jnp.float32) + fcb_ref[...]


def _final(out2, fcW, fcb):
    return pl.pallas_call(
        _final_body,
        out_shape=jax.ShapeDtypeStruct((1, 32), jnp.float32),
    )(out2, fcW.T, fcb.reshape(1, 32))


def _scA_body(src_hbm, dst_hbm, td_hbm, ts_hbm, msp_hbm,
              p_hbm, sp_hbm,
              td_v, ts_v, msp_v, src_v, dst_v, pall_v, sidx_v, s_sh):
    c = lax.axis_index("c")
    s = lax.axis_index("s")

    def _zb(i, _):
        pall_v[pl.ds(pl.multiple_of(i * 16, 16), 16)] = jnp.zeros((16,), jnp.float32)
        return 0
    lax.fori_loop(0, 4000 // 16, _zb, 0)

    @pl.when(s < 10)
    def _zero():
        pltpu.sync_copy(pall_v.at[pl.ds(0, 4000)],
                        s_sh.at[pl.ds(s * 4000, 4000)])

    pltpu.sync_copy(td_hbm, td_v)
    pltpu.sync_copy(ts_hbm, ts_v)
    pltpu.sync_copy(msp_hbm, msp_v)
    plsc.subcore_barrier()

    wid = c * SC_SUBCORES + s
    lane = lax.iota(jnp.int32, 16)

    def _chunk(k, _):
        base = wid * (NE // NSUB) + k * CE
        pltpu.sync_copy(src_hbm.at[pl.ds(base, CE)], src_v)
        pltpu.sync_copy(dst_hbm.at[pl.ds(base, CE)], dst_v)

        @plsc.parallel_loop(0, CE // 16, 1, unroll=2)
        def _vec(i):
            off = pl.multiple_of(i * 16, 16)
            d16 = dst_v[pl.ds(off, 16)]
            s16 = src_v[pl.ds(off, 16)]
            rows = off + lane
            for hh in range(NH):
                td16 = plsc.load_gather(td_v, [d16 * NH + hh])
                ts16 = plsc.load_gather(ts_v, [s16 * NH + hh])
                mh = msp_v[pl.ds(hh * 16, 16)]
                e = _lrelu(td16 + ts16)
                mp = _lrelu(td16 + mh)
                p = jnp.exp(e - mp)
                plsc.store_scatter(pall_v, [rows * NH + hh], p)
                plsc.store_scatter(sidx_v, [rows * NH + hh], d16 * NH + hh)

        pltpu.sync_copy(pall_v, p_hbm.at[pl.ds(base * NH, CE * NH)])
        pltpu.sync_copy(pall_v, s_sh.at[sidx_v], add=True)
        return 0
    lax.fori_loop(0, NE // NSUB // CE, _chunk, 0)

    plsc.subcore_barrier()

    @pl.when(s < 10)
    def _writeout():
        pltpu.sync_copy(s_sh.at[pl.ds(s * 4000, 4000)],
                        sp_hbm.at[pl.ds(c * (NN * NH) + s * 4000, 4000)])


_scA = functools.partial(
    pl.kernel,
    out_type=(
        jax.ShapeDtypeStruct((NE * NH,), jnp.float32),
        jax.ShapeDtypeStruct((SC_CORES * NN * NH,), jnp.float32),
    ),
    mesh=_mesh,
    compiler_params=pltpu.CompilerParams(
        needs_layout_passes=False, use_tc_tiling_on_sc=False),
    scratch_types=[
        pltpu.VMEM((NN * NH,), jnp.float32),
        pltpu.VMEM((NN * NH,), jnp.float32),
        pltpu.VMEM((NH * 16,), jnp.float32),
        pltpu.VMEM((CE,), jnp.int32),
        pltpu.VMEM((CE,), jnp.int32),
        pltpu.VMEM((CE * NH,), jnp.float32),
        pltpu.VMEM((CE * NH,), jnp.int32),
        pltpu.VMEM_SHARED((NN * NH,), jnp.float32),
    ],
)(_scA_body)


def _scN_body(dst_hbm, p_hbm, sp_hbm,
              w_hbm,
              r_v, sA_v, sB_v, dst_v, p_v):
    c = lax.axis_index("c")
    s = lax.axis_index("s")
    lane = lax.iota(jnp.int32, 16)

    for k in range(10):
        pltpu.sync_copy(sp_hbm.at[pl.ds(k * 4000, 4000)], sA_v)
        pltpu.sync_copy(sp_hbm.at[pl.ds(NN * NH + k * 4000, 4000)], sB_v)

        def _rv(i, _, k=k):
            off = pl.multiple_of(i * 16, 16)
            a = sA_v[pl.ds(off, 16)]
            b = sB_v[pl.ds(off, 16)]
            r_v[pl.ds(k * 4000 + off, 16)] = 1.0 / (a + b + 1e-16)
            return 0
        lax.fori_loop(0, 250, _rv, 0)

    wid = c * SC_SUBCORES + s

    def _chunk(k, _):
        base = wid * (NE // NSUB) + k * CE
        pltpu.sync_copy(dst_hbm.at[pl.ds(base, CE)], dst_v)
        pltpu.sync_copy(p_hbm.at[pl.ds(base * NH, CE * NH)], p_v)

        @plsc.parallel_loop(0, CE // 16, 1, unroll=2)
        def _vec(i):
            off = pl.multiple_of(i * 16, 16)
            d16 = dst_v[pl.ds(off, 16)]
            e16 = off + lane
            for hh in range(NH):
                p16 = plsc.load_gather(p_v, [e16 * NH + hh])
                r16 = plsc.load_gather(r_v, [d16 * NH + hh])
                plsc.store_scatter(p_v, [e16 * NH + hh], p16 * r16)

        pltpu.sync_copy(p_v, w_hbm.at[pl.ds(base * NH, CE * NH)])
        return 0
    lax.fori_loop(0, NE // NSUB // CE, _chunk, 0)


_scN = functools.partial(
    pl.kernel,
    out_type=jax.ShapeDtypeStruct((NE * NH,), jnp.float32),
    mesh=_mesh,
    compiler_params=pltpu.CompilerParams(
        needs_layout_passes=False, use_tc_tiling_on_sc=False),
    scratch_types=[
        pltpu.VMEM((NN * NH,), jnp.float32),
        pltpu.VMEM((4000,), jnp.float32),
        pltpu.VMEM((4000,), jnp.float32),
        pltpu.VMEM((CE,), jnp.int32),
        pltpu.VMEM((CE * NH,), jnp.float32),
    ],
)(_scN_body)


CB = 400
SB = 80


def _scB_body(fo, src_hbm, dst_hbm, w_hbm, wh_hbm,
              out_hbm,
              src_v, dst_v, w_v, w2_v, gA_v, gB_v, out_sh,
              gsemA, gsemB, ssemA, ssemB):
    c = lax.axis_index("c")
    s = lax.axis_index("s")
    lane = lax.iota(jnp.int32, 16)
    two_c = 2 * c
    nb = fo // 16
    gbuf = (gA_v, gB_v)
    gsem = (gsemA, gsemB)
    ssem = (ssemA, ssemB)

    @plsc.parallel_loop(0, SB, 1, unroll=4)
    def _zg(i):
        for f in range(nb):
            gA_v[i, pl.ds(f * 16, 16)] = jnp.zeros((16,), jnp.float32)
    for t in range(625 // SB):
        pltpu.sync_copy(gA_v, out_sh.at[pl.ds(s * 625 + t * SB, SB)])
    pltpu.sync_copy(gA_v.at[pl.ds(0, 625 % SB)],
                    out_sh.at[pl.ds(s * 625 + (625 // SB) * SB, 625 % SB)])
    plsc.subcore_barrier()

    nt = CB // SB

    def _chunk(k, _):
        eb = s * (NE // SC_SUBCORES) + k * CB
        pltpu.sync_copy(src_hbm.at[pl.ds(eb, CB)], src_v)
        pltpu.sync_copy(w_hbm.at[pl.ds(eb * NH, CB * NH)], w_v)
        for t in range(nt):
            pltpu.sync_copy(dst_hbm.at[pl.ds(eb + t * SB, SB)], dst_v.at[t])

        @plsc.parallel_loop(0, CB // 16, 1, unroll=2)
        def _adj(i):
            off = pl.multiple_of(i * 16, 16)
            src_v[pl.ds(off, 16)] = src_v[pl.ds(off, 16)] + c * NN

        @plsc.parallel_loop(0, CB // 16, 1, unroll=2)
        def _wv(i):
            off = pl.multiple_of(i * 16, 16)
            e16 = off + lane
            w0 = plsc.load_gather(w_v, [e16 * NH + two_c])
            w1 = plsc.load_gather(w_v, [e16 * NH + two_c + 1])
            plsc.store_scatter(w2_v, [e16 * 2], w0)
            plsc.store_scatter(w2_v, [e16 * 2 + 1], w1)

        def _issue_gather(t):
            p = t % 2
            return pltpu.async_copy(
                wh_hbm.at[src_v.at[pl.ds(t * SB, SB)]], gbuf[p], gsem[p])

        g_desc = _issue_gather(0)
        s_desc = [None, None]
        for t in range(nt):
            p = t % 2
            if t + 1 < nt:
                if s_desc[1 - p] is not None:
                    s_desc[1 - p].wait()
                    s_desc[1 - p] = None
                g_next = _issue_gather(t + 1)
            else:
                g_next = None
            g_desc.wait()

            @plsc.parallel_loop(0, SB, 1, unroll=4)
            def _scale(e, t=t, p=p):
                wv = w2_v[pl.ds(2 * (t * SB + e), 16)]
                w0 = wv[0]
                w1 = wv[1]
                for f in range(nb):
                    mf = w0 if f < nb // 2 else w1
                    gbuf[p][e, pl.ds(f * 16, 16)] = (
                        gbuf[p][e, pl.ds(f * 16, 16)] * mf)

            s_desc[p] = pltpu.async_copy(
                gbuf[p], out_sh.at[dst_v.at[t]], ssem[p], add=True)
            g_desc = g_next
        for p in range(2):
            if s_desc[p] is not None:
                s_desc[p].wait()
        return 0
    lax.fori_loop(0, NE // SC_SUBCORES // CB, _chunk, 0)

    plsc.subcore_barrier()

    @pl.when(s < 10)
    def _writeout():
        pltpu.sync_copy(out_sh.at[pl.ds(s * 1000, 1000)],
                        out_hbm.at[pl.ds(c * NN + s * 1000, 1000)])


def _make_scB(layer):
    fo = 128 if layer == 1 else 64
    return functools.partial(
        pl.kernel,
        out_type=jax.ShapeDtypeStruct((2 * NN, fo), jnp.float32),
        mesh=_mesh,
        compiler_params=pltpu.CompilerParams(
            needs_layout_passes=False, use_tc_tiling_on_sc=False),
        scratch_types=[
            pltpu.VMEM((CB,), jnp.int32),
            pltpu.VMEM((CB // SB, SB), jnp.int32),
            pltpu.VMEM((CB * NH,), jnp.float32),
            pltpu.VMEM((CB * 2 + 16,), jnp.float32),
            pltpu.VMEM((SB, fo), jnp.float32),
            pltpu.VMEM((SB, fo), jnp.float32),
            pltpu.VMEM_SHARED((NN, fo), jnp.float32),
            pltpu.SemaphoreType.DMA,
            pltpu.SemaphoreType.DMA,
            pltpu.SemaphoreType.DMA,
            pltpu.SemaphoreType.DMA,
        ],
    )(functools.partial(_scB_body, fo))


_scB1 = _make_scB(1)
_scB2 = _make_scB(2)


def _make_block_A(Aw, hdim):
    z = jnp.zeros((NH, hdim, NH), jnp.float32)
    idx = jnp.arange(NH)
    z = z.at[idx, :, idx].set(Aw[:, :hdim])
    return z.reshape(NH * hdim, NH)


def _make_block_A2(Aw, hdim):
    z = jnp.zeros((NH, hdim, NH), jnp.float32)
    idx = jnp.arange(NH)
    z = z.at[idx, :, idx].set(Aw[:, hdim:])
    return z.reshape(NH * hdim, NH)


@jax.jit
def kernel(h, edge_index, W1, b1, A1w, A1b, W2, b2, A2w, A2b, fcW, fcb):
    src = edge_index[0]
    dst = edge_index[1]

    w1catT = W1.reshape(NH * 64, 128).T
    b1cat = b1.reshape(NH * 64)
    Ad1 = _make_block_A(A1w, 64)
    As1 = _make_block_A2(A1w, 64)
    w2catT = W2.reshape(NH * 32, NH * 64).T
    b2cat = b2.reshape(NH * 32)
    Ad2 = _make_block_A(A2w, 32)
    As2 = _make_block_A2(A2w, 32)

    wh1, td1, ts1, msp1 = _prep1(h, w1catT, b1cat, Ad1, A1b, As1, 256)
    p1, sp1 = _scA(src, dst, td1.reshape(-1), ts1.reshape(-1),
                   msp1.reshape(-1))
    wh1cat = jnp.concatenate([wh1[:, :128], wh1[:, 128:]], axis=0)
    w1 = _scN(dst, p1, sp1)
    out1cat = _scB1(src, dst, w1, wh1cat)

    wh2, td2, ts2, msp2 = _prep2(out1cat, w2catT, b2cat, Ad2, A2b, As2, 128)
    p2, sp2 = _scA(src, dst, td2.reshape(-1), ts2.reshape(-1),
                   msp2.reshape(-1))
    wh2cat = jnp.concatenate([wh2[:, :64], wh2[:, 64:]], axis=0)
    w2 = _scN(dst, p2, sp2)
    out2cat = _scB2(src, dst, w2, wh2cat)

    return _final(out2cat, fcW, fcb)
